# Initial kernel scaffold; baseline (speedup 1.0000x reference)
import jax
import jax.numpy as jnp
from jax.experimental import pallas as pl


def kernel(session_day_of_week, session_index, W):
    raise NotImplementedError("write your pallas kernel here")


# speedup vs baseline: 3.5651x; 3.5651x over previous
import functools

import jax
import jax.numpy as jnp
from jax import lax
from jax.experimental import pallas as pl
from jax.experimental.pallas import tpu as pltpu, tpu_sc as plsc

NUM_SEEDS = 32
BATCH = 16384
IN_FEATURES = 7
NC, NS, L = 2, 16, 16
NW = NC * NS
B_PER_W = BATCH // NW
GROUPS = B_PER_W // L


def _sc_body(sdow_hbm, idx_hbm, w_hbm, out_hbm, idx_v, day_v, w_v, out_v, sem):
    wid = lax.axis_index("s") * NC + lax.axis_index("c")
    base = wid * B_PER_W

    pltpu.sync_copy(idx_hbm.at[pl.ds(base, B_PER_W)], idx_v)
    pltpu.async_copy(sdow_hbm.at[idx_v], day_v, sem).wait()
    pltpu.sync_copy(w_hbm, w_v)

    def group(g, _):
        day_vec = day_v[pl.ds(g * L, L)]
        for s in range(NUM_SEEDS):
            out_v[s, pl.ds(g * L, L)] = plsc.load_gather(
                w_v, [day_vec + (s * IN_FEATURES)]
            )
        return _

    lax.fori_loop(0, GROUPS, group, 0, unroll=False)

    pltpu.sync_copy(out_v, out_hbm.at[:, pl.ds(base, B_PER_W)])


@jax.jit
def kernel(session_day_of_week, session_index, W):
    mesh = plsc.VectorSubcoreMesh(
        core_axis_name="c", subcore_axis_name="s", num_cores=NC, num_subcores=NS
    )
    run = functools.partial(
        pl.kernel,
        out_type=jax.ShapeDtypeStruct((NUM_SEEDS, BATCH), jnp.float32),
        mesh=mesh,
        scratch_types=[
            pltpu.VMEM((B_PER_W,), jnp.int32),
            pltpu.VMEM((B_PER_W,), jnp.int32),
            pltpu.VMEM((NUM_SEEDS * IN_FEATURES,), jnp.float32),
            pltpu.VMEM((NUM_SEEDS, B_PER_W), jnp.float32),
            pltpu.SemaphoreType.DMA,
        ],
        compiler_params=pltpu.CompilerParams(needs_layout_passes=False),
    )(_sc_body)
    return run(
        session_day_of_week.astype(jnp.int32),
        session_index.astype(jnp.int32),
        W.reshape(-1),
    )


# speedup vs baseline: 3.9290x; 1.1021x over previous
import functools

import jax
import jax.numpy as jnp
from jax import lax
from jax.experimental import pallas as pl
from jax.experimental.pallas import tpu as pltpu, tpu_sc as plsc

NUM_SEEDS = 32
BATCH = 16384
IN_FEATURES = 7
NC, NS, L = 2, 16, 16
NW = NC * NS
B_PER_W = BATCH // NW
GROUPS = B_PER_W // L


def _sc_body(sdow_hbm, idx_hbm, w_hbm, out_hbm, idx_v, day_v, w_v, out_v, sem):
    wid = lax.axis_index("s") * NC + lax.axis_index("c")
    base = wid * B_PER_W

    pltpu.sync_copy(idx_hbm.at[pl.ds(base, B_PER_W)], idx_v)
    pltpu.async_copy(sdow_hbm.at[idx_v], day_v, sem).wait()
    pltpu.sync_copy(w_hbm, w_v)

    @plsc.parallel_loop(0, GROUPS)
    def group(g):
        day_vec = day_v[pl.ds(g * L, L)]
        for s in range(NUM_SEEDS):
            out_v[s, pl.ds(g * L, L)] = plsc.load_gather(w_v.at[s], [day_vec])

    pltpu.sync_copy(out_v, out_hbm.at[:, pl.ds(base, B_PER_W)])


@jax.jit
def kernel(session_day_of_week, session_index, W):
    mesh = plsc.VectorSubcoreMesh(
        core_axis_name="c", subcore_axis_name="s", num_cores=NC, num_subcores=NS
    )
    run = functools.partial(
        pl.kernel,
        out_type=jax.ShapeDtypeStruct((NUM_SEEDS, BATCH), jnp.float32),
        mesh=mesh,
        scratch_types=[
            pltpu.VMEM((B_PER_W,), jnp.int32),
            pltpu.VMEM((B_PER_W,), jnp.int32),
            pltpu.VMEM((NUM_SEEDS, 8), jnp.float32),
            pltpu.VMEM((NUM_SEEDS, B_PER_W), jnp.float32),
            pltpu.SemaphoreType.DMA,
        ],
        compiler_params=pltpu.CompilerParams(needs_layout_passes=False),
    )(_sc_body)
    return run(
        session_day_of_week.astype(jnp.int32),
        session_index.astype(jnp.int32),
        jnp.pad(W, ((0, 0), (0, 8 - IN_FEATURES))),
    )


# speedup vs baseline: 3.9325x; 1.0009x over previous
import functools

import jax
import jax.numpy as jnp
from jax import lax
from jax.experimental import pallas as pl
from jax.experimental.pallas import tpu as pltpu, tpu_sc as plsc

NUM_SEEDS = 32
BATCH = 16384
IN_FEATURES = 7
NC, NS, L = 2, 16, 16
NW = NC * NS
B_PER_W = BATCH // NW
GROUPS = B_PER_W // L


def _sc_body(sdow_hbm, idx_hbm, w_hbm, out_hbm, idx_v, day_v, w_v, out_v, sem):
    wid = lax.axis_index("s") * NC + lax.axis_index("c")
    base = wid * B_PER_W

    pltpu.sync_copy(idx_hbm.at[pl.ds(base, B_PER_W)], idx_v)
    pltpu.async_copy(sdow_hbm.at[idx_v], day_v, sem).wait()
    pltpu.sync_copy(w_hbm, w_v)

    @plsc.parallel_loop(0, GROUPS)
    def group(g):
        day_vec = day_v[pl.ds(g * L, L)]
        for s in range(NUM_SEEDS):
            out_v[s, pl.ds(g * L, L)] = plsc.load_gather(w_v.at[s], [day_vec])

    pltpu.sync_copy(out_v, out_hbm.at[:, pl.ds(base, B_PER_W)])


@jax.jit
def kernel(session_day_of_week, session_index, W):
    mesh = plsc.VectorSubcoreMesh(
        core_axis_name="c", subcore_axis_name="s", num_cores=NC, num_subcores=NS
    )
    run = functools.partial(
        pl.kernel,
        out_type=jax.ShapeDtypeStruct((NUM_SEEDS, BATCH), jnp.float32),
        mesh=mesh,
        scratch_types=[
            pltpu.VMEM((B_PER_W,), jnp.int32),
            pltpu.VMEM((B_PER_W,), jnp.int32),
            pltpu.VMEM((NUM_SEEDS, IN_FEATURES), jnp.float32),
            pltpu.VMEM((NUM_SEEDS, B_PER_W), jnp.float32),
            pltpu.SemaphoreType.DMA,
        ],
        compiler_params=pltpu.CompilerParams(needs_layout_passes=False),
    )(_sc_body)
    return run(
        session_day_of_week.astype(jnp.int32),
        session_index.astype(jnp.int32),
        W,
    )
